# baseline (speedup 1.0000x reference)
import functools

import jax
import jax.numpy as jnp
from jax import lax
from jax.experimental import pallas as pl
from jax.experimental.pallas import tpu as pltpu
from jax.experimental.pallas import tpu_sc as plsc

_ROWS, _COLS = 16384, 200
_OCOLS = 2 * _COLS
_NC, _NS = 2, 16
_NW = _NC * _NS
_RPW = _ROWS // _NW
_CH = 32
_NCHUNK = _RPW // _CH


def _sc_body(ts_hbm, lab_hbm, out_hbm, t_buf, l_buf, o_buf, sem_t, sem_l, sem_o):
    wid = lax.axis_index("s") * _NC + lax.axis_index("c")
    base_row = wid * _RPW
    iota = lax.iota(jnp.int32, 16)

    def chunk(ci, carry):
        r0 = base_row + ci * _CH
        ct = pltpu.make_async_copy(
            ts_hbm.at[pl.ds(r0, _CH), :], t_buf, sem_t)
        cl = pltpu.make_async_copy(
            lab_hbm.at[pl.ds(r0, _CH), :], l_buf, sem_l)
        ct.start()
        cl.start()
        ct.wait()
        cl.wait()

        def row(r, rcarry):
            rbase = r * _OCOLS
            for off in [16 * w for w in range(12)] + [_COLS - 16]:
                src_c = iota + off
                idx_t = 2 * src_c + rbase
                idx_l = idx_t + 1
                vt = t_buf[r, pl.ds(off, 16)]
                plsc.store_scatter(o_buf, [idx_t], vt)
                vl = l_buf[r, pl.ds(off, 16)]
                plsc.store_scatter(o_buf, [idx_l], vl)
            return rcarry

        lax.fori_loop(0, _CH, row, 0)
        co = pltpu.make_async_copy(
            o_buf, out_hbm.at[pl.ds(r0 * _OCOLS, _CH * _OCOLS)], sem_o)
        co.start()
        co.wait()
        return carry

    lax.fori_loop(0, _NCHUNK, chunk, 0)


def kernel(timestamps, labels, seq_lens):
    mesh = plsc.VectorSubcoreMesh(
        core_axis_name="c", subcore_axis_name="s",
        num_cores=_NC, num_subcores=_NS)
    flat = pl.kernel(
        _sc_body,
        out_type=jax.ShapeDtypeStruct((_ROWS * _OCOLS,), timestamps.dtype),
        mesh=mesh,
        compiler_params=pltpu.CompilerParams(
            use_tc_tiling_on_sc=True, needs_layout_passes=False),
        scratch_types=[
            pltpu.VMEM((_CH, _COLS), jnp.float32),
            pltpu.VMEM((_CH, _COLS), jnp.float32),
            pltpu.VMEM((_CH * _OCOLS,), jnp.float32),
            pltpu.SemaphoreType.DMA,
            pltpu.SemaphoreType.DMA,
            pltpu.SemaphoreType.DMA,
        ],
    )(timestamps, labels)
    payload = flat.reshape(_ROWS, _COLS, 2)
    return (payload, seq_lens)


# speedup vs baseline: 11.5814x; 11.5814x over previous
import functools

import jax
import jax.numpy as jnp
from jax import lax
from jax.experimental import pallas as pl
from jax.experimental.pallas import tpu as pltpu
from jax.experimental.pallas import tpu_sc as plsc

_ROWS, _COLS = 16384, 200
_NC, _NS = 2, 16
_NW = _NC * _NS
_TILE = 128
_TPW = _ROWS // _NW // _TILE
_JSTR = 2 * _TILE
_OUT_JSTR = _ROWS * 2 // _TILE * _TILE


def _sc_body(ts_hbm, lab_hbm, out_hbm, t_buf, l_buf, o_buf, sem_t, sem_l, sem_o):
    wid = lax.axis_index("s") * _NC + lax.axis_index("c")
    iota = lax.iota(jnp.int32, 16)

    def chunk(ci, carry):
        it = wid * _TPW + ci
        i0 = it * _TILE
        ct = pltpu.make_async_copy(
            ts_hbm.at[pl.ds(i0, _TILE), :], t_buf, sem_t)
        cl = pltpu.make_async_copy(
            lab_hbm.at[pl.ds(i0, _TILE), :], l_buf, sem_l)
        ct.start()
        cl.start()
        ct.wait()
        cl.wait()

        def row(r, rcarry):
            for off in [16 * w for w in range(12)] + [_COLS - 16]:
                idx = (iota + off) * _JSTR + r
                vt = t_buf[r, pl.ds(off, 16)]
                plsc.store_scatter(o_buf, [idx], vt)
                vl = l_buf[r, pl.ds(off, 16)]
                plsc.store_scatter(o_buf, [idx + _TILE], vl)
            return rcarry

        lax.fori_loop(0, _TILE, row, 0)

        obase = it * _JSTR

        def out_start(j, jcarry):
            pltpu.make_async_copy(
                o_buf.at[pl.ds(j * _JSTR, _JSTR)],
                out_hbm.at[pl.ds(j * _OUT_JSTR + obase, _JSTR)],
                sem_o).start()
            return jcarry

        def out_wait(j, jcarry):
            pltpu.make_async_copy(
                o_buf.at[pl.ds(j * _JSTR, _JSTR)],
                out_hbm.at[pl.ds(j * _OUT_JSTR + obase, _JSTR)],
                sem_o).wait()
            return jcarry

        lax.fori_loop(0, _COLS, out_start, 0)
        lax.fori_loop(0, _COLS, out_wait, 0)
        return carry

    lax.fori_loop(0, _TPW, chunk, 0)


def kernel(timestamps, labels, seq_lens):
    mesh = plsc.VectorSubcoreMesh(
        core_axis_name="c", subcore_axis_name="s",
        num_cores=_NC, num_subcores=_NS)
    flat = pl.kernel(
        _sc_body,
        out_type=jax.ShapeDtypeStruct((_ROWS * 2 * _COLS,), timestamps.dtype),
        mesh=mesh,
        compiler_params=pltpu.CompilerParams(
            use_tc_tiling_on_sc=True, needs_layout_passes=False),
        scratch_types=[
            pltpu.VMEM((_TILE, _COLS), jnp.float32),
            pltpu.VMEM((_TILE, _COLS), jnp.float32),
            pltpu.VMEM((_COLS * _JSTR,), jnp.float32),
            pltpu.SemaphoreType.DMA,
            pltpu.SemaphoreType.DMA,
            pltpu.SemaphoreType.DMA,
        ],
    )(timestamps, labels)
    payload = (flat.reshape(_COLS, _ROWS // _TILE, 2, _TILE)
               .transpose(1, 3, 0, 2)
               .reshape(_ROWS, _COLS, 2))
    return (payload, seq_lens)


# speedup vs baseline: 32.3842x; 2.7962x over previous
import functools

import jax
import jax.numpy as jnp
from jax import lax
from jax.experimental import pallas as pl
from jax.experimental.pallas import tpu as pltpu
from jax.experimental.pallas import tpu_sc as plsc

_ROWS, _COLS = 16384, 200
_NC, _NS = 2, 16
_NW = _NC * _NS
_TILE = 128
_TPW = _ROWS // _NW // _TILE
_JSTR = 2 * _TILE
_OUT_JSTR = _ROWS * 2
_OSZ = _COLS * _JSTR


def _sc_body(ts_hbm, lab_hbm, out_hbm, t_buf, l_buf, o_buf, sem_t, sem_l, sem_o):
    wid = lax.axis_index("s") * _NC + lax.axis_index("c")
    iota = lax.iota(jnp.int32, 16)
    o_hi = o_buf.at[pl.ds(_TILE, _OSZ - _TILE)]

    def in_copies(ci):
        i0 = (wid * _TPW + ci) * _TILE
        return (
            pltpu.make_async_copy(
                ts_hbm.at[pl.ds(i0, _TILE), :], t_buf, sem_t),
            pltpu.make_async_copy(
                lab_hbm.at[pl.ds(i0, _TILE), :], l_buf, sem_l),
        )

    def start_in(ci):
        ct, cl = in_copies(ci)
        ct.start()
        cl.start()

    start_in(0)

    def chunk(ci, carry):
        ct, cl = in_copies(ci)
        ct.wait()
        cl.wait()

        @functools.partial(plsc.parallel_loop, 0, _TILE, unroll=2)
        def row(r):
            for off in [16 * w for w in range(12)] + [_COLS - 16]:
                idx = (iota + off) * _JSTR + r
                vt = t_buf[r, pl.ds(off, 16)]
                plsc.store_scatter(o_buf, [idx], vt)
                vl = l_buf[r, pl.ds(off, 16)]
                plsc.store_scatter(o_hi, [idx], vl)

        obase = (wid * _TPW + ci) * _JSTR

        def out_copy(j):
            return pltpu.make_async_copy(
                o_buf.at[pl.ds(j * _JSTR, _JSTR)],
                out_hbm.at[pl.ds(j * _OUT_JSTR + obase, _JSTR)],
                sem_o)

        def out_start(j, jcarry):
            out_copy(j).start()
            return jcarry

        lax.fori_loop(0, _COLS, out_start, 0)

        @pl.when(ci + 1 < _TPW)
        def _():
            start_in(ci + 1)

        def out_wait(j, jcarry):
            out_copy(j).wait()
            return jcarry

        lax.fori_loop(0, _COLS, out_wait, 0)
        return carry

    lax.fori_loop(0, _TPW, chunk, 0)


def kernel(timestamps, labels, seq_lens):
    mesh = plsc.VectorSubcoreMesh(
        core_axis_name="c", subcore_axis_name="s",
        num_cores=_NC, num_subcores=_NS)
    flat = pl.kernel(
        _sc_body,
        out_type=jax.ShapeDtypeStruct((_ROWS * 2 * _COLS,), timestamps.dtype),
        mesh=mesh,
        compiler_params=pltpu.CompilerParams(
            use_tc_tiling_on_sc=True, needs_layout_passes=False),
        scratch_types=[
            pltpu.VMEM((_TILE, _COLS), jnp.float32),
            pltpu.VMEM((_TILE, _COLS), jnp.float32),
            pltpu.VMEM((_OSZ,), jnp.float32),
            pltpu.SemaphoreType.DMA,
            pltpu.SemaphoreType.DMA,
            pltpu.SemaphoreType.DMA,
        ],
    )(timestamps, labels)
    payload = (flat.reshape(_COLS, _ROWS // _TILE, 2, _TILE)
               .transpose(1, 3, 0, 2)
               .reshape(_ROWS, _COLS, 2))
    return (payload, seq_lens)


# speedup vs baseline: 32.4533x; 1.0021x over previous
import functools

import jax
import jax.numpy as jnp
from jax import lax
from jax.experimental import pallas as pl
from jax.experimental.pallas import tpu as pltpu
from jax.experimental.pallas import tpu_sc as plsc

_ROWS, _COLS = 16384, 200
_NC, _NS = 2, 16
_NW = _NC * _NS
_TILE = 128
_TPW = _ROWS // _NW // _TILE
_JSTR = 2 * _TILE
_OUT_JSTR = _ROWS * 2
_OSZ = _COLS * _JSTR


def _sc_body(ts_hbm, lab_hbm, out_hbm, t_buf, l_buf, o_buf, sem_t, sem_l, sem_o):
    wid = lax.axis_index("s") * _NC + lax.axis_index("c")
    iota = lax.iota(jnp.int32, 16)
    o_hi = o_buf.at[pl.ds(_TILE, _OSZ - _TILE)]

    def in_copies(ci):
        i0 = (wid * _TPW + ci) * _TILE
        return (
            pltpu.make_async_copy(
                ts_hbm.at[pl.ds(i0, _TILE), :], t_buf, sem_t),
            pltpu.make_async_copy(
                lab_hbm.at[pl.ds(i0, _TILE), :], l_buf, sem_l),
        )

    def start_in(ci):
        ct, cl = in_copies(ci)
        ct.start()
        cl.start()

    start_in(0)

    def chunk(ci, carry):
        ct, cl = in_copies(ci)
        ct.wait()
        cl.wait()

        @functools.partial(plsc.parallel_loop, 0, _TILE, unroll=2)
        def row(r):
            for off in [16 * w for w in range(12)] + [_COLS - 16]:
                idx = (iota + off) * _JSTR + r
                vt = t_buf[r, pl.ds(off, 16)]
                plsc.store_scatter(o_buf, [idx], vt)
                vl = l_buf[r, pl.ds(off, 16)]
                plsc.store_scatter(o_hi, [idx], vl)

        obase = (wid * _TPW + ci) * _JSTR

        def out_copy(j):
            return pltpu.make_async_copy(
                o_buf.at[pl.ds(j * _JSTR, _JSTR)],
                out_hbm.at[pl.ds(j * _OUT_JSTR + obase, _JSTR)],
                sem_o)

        def out_start(j, jcarry):
            out_copy(j).start()
            return jcarry

        lax.fori_loop(0, _COLS, out_start, 0)

        @pl.when(ci + 1 < _TPW)
        def _():
            start_in(ci + 1)

        def out_wait(j, jcarry):
            out_copy(j).wait()
            return jcarry

        lax.fori_loop(0, _COLS, out_wait, 0)
        return carry

    lax.fori_loop(0, _TPW, chunk, 0)


def kernel(timestamps, labels, seq_lens):
    mesh = plsc.VectorSubcoreMesh(
        core_axis_name="c", subcore_axis_name="s",
        num_cores=_NC, num_subcores=_NS)
    flat = pl.kernel(
        _sc_body,
        out_type=jax.ShapeDtypeStruct((_ROWS * 2 * _COLS,), timestamps.dtype),
        mesh=mesh,
        compiler_params=pltpu.CompilerParams(
            use_tc_tiling_on_sc=True),
        scratch_types=[
            pltpu.VMEM((_TILE, _COLS), jnp.float32),
            pltpu.VMEM((_TILE, _COLS), jnp.float32),
            pltpu.VMEM((_OSZ,), jnp.float32),
            pltpu.SemaphoreType.DMA,
            pltpu.SemaphoreType.DMA,
            pltpu.SemaphoreType.DMA,
        ],
    )(timestamps, labels)
    payload = (flat.reshape(_COLS, _ROWS // _TILE, 2, _TILE)
               .transpose(1, 3, 0, 2)
               .reshape(_ROWS, _COLS, 2))
    return (payload, seq_lens)


# speedup vs baseline: 55.0678x; 1.6968x over previous
import functools

import jax
import jax.numpy as jnp
from jax import lax
from jax.experimental import pallas as pl
from jax.experimental.pallas import tpu as pltpu
from jax.experimental.pallas import tpu_sc as plsc

_ROWS, _COLS = 16384, 200
_NC, _NS = 2, 16
_NW = _NC * _NS
_TILE = 128
_TPW = _ROWS // _NW // _TILE
_JSTR = 2 * _TILE
_OUT_JSTR = _ROWS * 2
_OSZ = _COLS * _JSTR


def _sc_body(ts_hbm, lab_hbm, out_hbm, t_buf, l_buf, o_buf, sem_t, sem_l, sem_o):
    wid = lax.axis_index("s") * _NC + lax.axis_index("c")

    def in_copies(ci):
        i0 = (wid * _TPW + ci) * _TILE
        return (
            pltpu.make_async_copy(
                ts_hbm.at[:, pl.ds(i0, _TILE)], t_buf, sem_t),
            pltpu.make_async_copy(
                lab_hbm.at[:, pl.ds(i0, _TILE)], l_buf, sem_l),
        )

    def start_in(ci):
        ct, cl = in_copies(ci)
        ct.start()
        cl.start()

    start_in(0)

    def chunk(ci, carry):
        ct, cl = in_copies(ci)
        ct.wait()
        cl.wait()

        @functools.partial(plsc.parallel_loop, 0, _COLS, unroll=2)
        def row(j):
            base = j * _JSTR
            for w in range(_TILE // 16):
                off = 16 * w
                o_buf[pl.ds(base + off, 16)] = t_buf[j, pl.ds(off, 16)]
                o_buf[pl.ds(base + _TILE + off, 16)] = l_buf[j, pl.ds(off, 16)]

        obase = (wid * _TPW + ci) * _JSTR

        def out_copy(j):
            return pltpu.make_async_copy(
                o_buf.at[pl.ds(j * _JSTR, _JSTR)],
                out_hbm.at[pl.ds(j * _OUT_JSTR + obase, _JSTR)],
                sem_o)

        def out_start(j, jcarry):
            out_copy(j).start()
            return jcarry

        lax.fori_loop(0, _COLS, out_start, 0)

        @pl.when(ci + 1 < _TPW)
        def _():
            start_in(ci + 1)

        def out_wait(j, jcarry):
            out_copy(j).wait()
            return jcarry

        lax.fori_loop(0, _COLS, out_wait, 0)
        return carry

    lax.fori_loop(0, _TPW, chunk, 0)


def kernel(timestamps, labels, seq_lens):
    mesh = plsc.VectorSubcoreMesh(
        core_axis_name="c", subcore_axis_name="s",
        num_cores=_NC, num_subcores=_NS)
    flat = pl.kernel(
        _sc_body,
        out_type=jax.ShapeDtypeStruct((_ROWS * 2 * _COLS,), timestamps.dtype),
        mesh=mesh,
        compiler_params=pltpu.CompilerParams(
            use_tc_tiling_on_sc=True, needs_layout_passes=False),
        scratch_types=[
            pltpu.VMEM((_COLS, _TILE), jnp.float32),
            pltpu.VMEM((_COLS, _TILE), jnp.float32),
            pltpu.VMEM((_OSZ,), jnp.float32),
            pltpu.SemaphoreType.DMA,
            pltpu.SemaphoreType.DMA,
            pltpu.SemaphoreType.DMA,
        ],
    )(timestamps.T, labels.T)
    payload = (flat.reshape(_COLS, _ROWS // _TILE, 2, _TILE)
               .transpose(1, 3, 0, 2)
               .reshape(_ROWS, _COLS, 2))
    return (payload, seq_lens)
